# baseline (speedup 1.0000x reference)
import jax
import jax.numpy as jnp
from jax import lax
from jax.experimental import pallas as pl
from jax.experimental.pallas import tpu as pltpu
from jax.experimental.pallas import tpu_sc as plsc

V = 1000
VP = 1024
NB, NT = 1024, 50
N = NB * NT
NC, NS, L = 2, 16, 16
NW = NC * NS
NH = 2
H = N // NH
PER_W = H // NW
CH = 16
NCHUNK = PER_W // CH
NPAIR = NCHUNK // 2 - 1
NCT = V // 128
REM = V - NCT * 128


def _lse_body(table_ref, lse_ref):
    t = table_ref[...]
    m = jnp.max(t, axis=1, keepdims=True)
    s = jnp.sum(jnp.exp(t - m), axis=1, keepdims=True)
    lse_ref[...] = m + jnp.log(s)


def _compute_lse(table):
    return pl.pallas_call(
        _lse_body,
        out_shape=jax.ShapeDtypeStruct((V, 1), jnp.float32),
    )(table)


def _sc_body(table_hbm, idx_hbm, tgt_hbm, lse_hbm,
             out_hbm, part_hbm,
             idx_v, tgt_v, lse_v, rows_a, rows_b, acc_v, sem_a, sem_b,
             sem_s):
    wid = lax.axis_index("s") * NC + lax.axis_index("c")
    base = wid * PER_W
    pltpu.sync_copy(idx_hbm.at[pl.ds(base, PER_W)], idx_v)
    pltpu.sync_copy(tgt_hbm.at[pl.ds(base, PER_W)], tgt_v)
    pltpu.sync_copy(lse_hbm, lse_v)
    acc_v[...] = jnp.zeros((L,), jnp.float32)

    def gstart(c, rows_v, sem):
        pltpu.async_copy(table_hbm.at[idx_v.at[pl.ds(c * CH, CH)]], rows_v,
                         sem)

    def gwait(c, rows_v, sem):
        pltpu.make_async_copy(table_hbm.at[idx_v.at[pl.ds(c * CH, CH)]],
                              rows_v, sem).wait()

    def scat(c, rows_v):
        r0 = base + c * CH
        descs = []
        for g in range(CH // 8):
            descs.append(pltpu.async_copy(
                rows_v.at[pl.ds(g * 8, 8), pl.ds(0, NCT * 128)],
                out_hbm.at[pl.ds(r0 + g * 8, 8), pl.ds(0, NCT * 128)],
                sem_s))
            for k in range(8):
                descs.append(pltpu.async_copy(
                    rows_v.at[g * 8 + k, pl.ds(NCT * 128, REM)],
                    out_hbm.at[r0 + g * 8 + k, pl.ds(NCT * 128, REM)],
                    sem_s))
        return descs

    def extract(c, rows_v):
        off = c * CH

        def sub(i, carry):
            o2 = off + i * L
            rid = lax.iota(jnp.int32, L) + i * L
            tg = tgt_v[pl.ds(o2, L)]
            ix = idx_v[pl.ds(o2, L)]
            vals = plsc.load_gather(rows_v, [rid, tg])
            lses = plsc.load_gather(lse_v, [ix])
            acc_v[...] = acc_v[...] + (lses - vals)
            return carry

        lax.fori_loop(0, CH // L, sub, 0)

    def drain(descs):
        for d in descs:
            d.wait()

    gstart(0, rows_a, sem_a)

    def pair_body(i, carry):
        ca = 2 * i
        gstart(ca + 1, rows_b, sem_b)
        gwait(ca, rows_a, sem_a)
        da = scat(ca, rows_a)
        extract(ca, rows_a)
        drain(da)
        gstart(ca + 2, rows_a, sem_a)
        gwait(ca + 1, rows_b, sem_b)
        db = scat(ca + 1, rows_b)
        extract(ca + 1, rows_b)
        drain(db)
        return carry

    lax.fori_loop(0, NPAIR, pair_body, 0)

    ca = NCHUNK - 2
    gstart(ca + 1, rows_b, sem_b)
    gwait(ca, rows_a, sem_a)
    da = scat(ca, rows_a)
    extract(ca, rows_a)
    drain(da)
    gwait(ca + 1, rows_b, sem_b)
    db = scat(ca + 1, rows_b)
    extract(ca + 1, rows_b)
    drain(db)
    pltpu.sync_copy(acc_v, part_hbm.at[pl.ds(wid * L, L)])


def _sc_gather(table3, idx_flat, tgt_flat, lse):
    mesh = plsc.VectorSubcoreMesh(core_axis_name="c", subcore_axis_name="s")
    f = pl.kernel(
        _sc_body, mesh=mesh,
        compiler_params=pltpu.CompilerParams(use_tc_tiling_on_sc=True,
                                             needs_layout_passes=False),
        out_type=[
            jax.ShapeDtypeStruct((H, V), jnp.float32),
            jax.ShapeDtypeStruct((NW * L,), jnp.float32),
        ],
        scratch_types=[
            pltpu.VMEM((PER_W,), jnp.int32),
            pltpu.VMEM((PER_W,), jnp.int32),
            pltpu.VMEM((V,), jnp.float32),
            pltpu.VMEM((CH, VP), jnp.float32),
            pltpu.VMEM((CH, VP), jnp.float32),
            pltpu.VMEM((L,), jnp.float32),
            pltpu.SemaphoreType.DMA,
            pltpu.SemaphoreType.DMA,
            pltpu.SemaphoreType.DMA,
        ],
    )
    return f(table3, idx_flat, tgt_flat, lse)


TB = 512


def _tr_body(in_ref, out_ref):
    out_ref[...] = jnp.transpose(in_ref[...], (1, 0))


def _tr_body_alias(prev_ref, in_ref, out_ref):
    del prev_ref
    out_ref[...] = jnp.transpose(in_ref[...], (1, 0))


def _transpose_half(prev, half_rm, k):
    base = k * H // TB
    out_spec = pl.BlockSpec((V, TB), lambda g, b=base: (0, b + g))
    out_shape = jax.ShapeDtypeStruct((V, N), jnp.float32)
    if prev is None:
        return pl.pallas_call(
            _tr_body,
            grid=(H // TB,),
            in_specs=[pl.BlockSpec((TB, V), lambda g: (g, 0))],
            out_specs=out_spec,
            out_shape=out_shape,
        )(half_rm)
    return pl.pallas_call(
        _tr_body_alias,
        grid=(H // TB,),
        in_specs=[pl.BlockSpec(memory_space=pl.ANY),
                  pl.BlockSpec((TB, V), lambda g: (g, 0))],
        out_specs=out_spec,
        out_shape=out_shape,
        input_output_aliases={0: 0},
    )(prev, half_rm)


def _red_body(part_ref, out_ref):
    out_ref[...] = jnp.sum(part_ref[...], keepdims=True).reshape(1, 1) * (
        1.0 / N)


def _reduce_loss(part):
    return pl.pallas_call(
        _red_body,
        out_shape=jax.ShapeDtypeStruct((1, 1), jnp.float32),
    )(part.reshape(NH * 4, 128))


def kernel(idx, targets, table):
    idx_flat = idx.reshape(N).astype(jnp.int32)
    tgt_flat = targets.reshape(N).astype(jnp.int32)
    table3 = jnp.pad(table, ((0, 0), (0, VP - V)))
    lse = _compute_lse(table).reshape(V)
    halves = []
    parts = []
    for k in range(NH):
        half_rm, p = _sc_gather(table3, idx_flat[k * H:(k + 1) * H],
                                tgt_flat[k * H:(k + 1) * H], lse)
        halves.append(half_rm)
        parts.append(p)
    acc = _transpose_half(None, halves[0], 0)
    for k in range(1, NH):
        acc = _transpose_half(acc, halves[k], k)
    logits_flat = jnp.transpose(acc, (1, 0))
    loss = _reduce_loss(jnp.concatenate(parts))[0, 0]
    return (logits_flat, loss)


# speedup vs baseline: 1.0447x; 1.0447x over previous
import jax
import jax.numpy as jnp
from jax import lax
from jax.experimental import pallas as pl
from jax.experimental.pallas import tpu as pltpu
from jax.experimental.pallas import tpu_sc as plsc

V = 1000
VP = 1024
NB, NT = 1024, 50
N = NB * NT
NC, NS, L = 2, 16, 16
NW = NC * NS
NH = 1
H = N // NH
PER_W = H // NW
CH = 32
NCHUNK = PER_W // CH
NPAIR = NCHUNK // 2 - 1
NCT = V // 128
REM = V - NCT * 128


def _lse_body(table_ref, lse_ref):
    t = table_ref[...]
    m = jnp.max(t, axis=1, keepdims=True)
    s = jnp.sum(jnp.exp(t - m), axis=1, keepdims=True)
    lse_ref[...] = m + jnp.log(s)


def _compute_lse(table):
    return pl.pallas_call(
        _lse_body,
        out_shape=jax.ShapeDtypeStruct((V, 1), jnp.float32),
    )(table)


def _sc_body(table_hbm, idx_hbm, tgt_hbm, lse_hbm,
             out_hbm, part_hbm,
             idx_v, tgt_v, lse_v, rows_a, rows_b, acc_v, sem_a, sem_b,
             sem_s):
    wid = lax.axis_index("s") * NC + lax.axis_index("c")
    base = wid * PER_W
    pltpu.sync_copy(idx_hbm.at[pl.ds(base, PER_W)], idx_v)
    pltpu.sync_copy(tgt_hbm.at[pl.ds(base, PER_W)], tgt_v)
    pltpu.sync_copy(lse_hbm, lse_v)
    acc_v[...] = jnp.zeros((L,), jnp.float32)

    def gstart(c, rows_v, sem):
        pltpu.async_copy(table_hbm.at[idx_v.at[pl.ds(c * CH, CH)]], rows_v,
                         sem)

    def gwait(c, rows_v, sem):
        pltpu.make_async_copy(table_hbm.at[idx_v.at[pl.ds(c * CH, CH)]],
                              rows_v, sem).wait()

    def scat(c, rows_v):
        r0 = base + c * CH
        descs = []
        for g in range(CH // 8):
            descs.append(pltpu.async_copy(
                rows_v.at[pl.ds(g * 8, 8), pl.ds(0, NCT * 128)],
                out_hbm.at[pl.ds(r0 + g * 8, 8), pl.ds(0, NCT * 128)],
                sem_s))
            for k in range(8):
                descs.append(pltpu.async_copy(
                    rows_v.at[g * 8 + k, pl.ds(NCT * 128, REM)],
                    out_hbm.at[r0 + g * 8 + k, pl.ds(NCT * 128, REM)],
                    sem_s))
        return descs

    def extract(c, rows_v):
        off = c * CH

        def sub(i, carry):
            o2 = off + i * L
            rid = lax.iota(jnp.int32, L) + i * L
            tg = tgt_v[pl.ds(o2, L)]
            ix = idx_v[pl.ds(o2, L)]
            vals = plsc.load_gather(rows_v, [rid, tg])
            lses = plsc.load_gather(lse_v, [ix])
            acc_v[...] = acc_v[...] + (lses - vals)
            return carry

        lax.fori_loop(0, CH // L, sub, 0)

    def drain(descs):
        for d in descs:
            d.wait()

    gstart(0, rows_a, sem_a)

    def pair_body(i, carry):
        ca = 2 * i
        gstart(ca + 1, rows_b, sem_b)
        gwait(ca, rows_a, sem_a)
        da = scat(ca, rows_a)
        extract(ca, rows_a)
        drain(da)
        gstart(ca + 2, rows_a, sem_a)
        gwait(ca + 1, rows_b, sem_b)
        db = scat(ca + 1, rows_b)
        extract(ca + 1, rows_b)
        drain(db)
        return carry

    lax.fori_loop(0, NPAIR, pair_body, 0)

    ca = NCHUNK - 2
    gstart(ca + 1, rows_b, sem_b)
    gwait(ca, rows_a, sem_a)
    da = scat(ca, rows_a)
    extract(ca, rows_a)
    drain(da)
    gwait(ca + 1, rows_b, sem_b)
    db = scat(ca + 1, rows_b)
    extract(ca + 1, rows_b)
    drain(db)
    pltpu.sync_copy(acc_v, part_hbm.at[pl.ds(wid * L, L)])


def _sc_gather(table3, idx_flat, tgt_flat, lse):
    mesh = plsc.VectorSubcoreMesh(core_axis_name="c", subcore_axis_name="s")
    f = pl.kernel(
        _sc_body, mesh=mesh,
        compiler_params=pltpu.CompilerParams(use_tc_tiling_on_sc=True,
                                             needs_layout_passes=False),
        out_type=[
            jax.ShapeDtypeStruct((H, V), jnp.float32),
            jax.ShapeDtypeStruct((NW * L,), jnp.float32),
        ],
        scratch_types=[
            pltpu.VMEM((PER_W,), jnp.int32),
            pltpu.VMEM((PER_W,), jnp.int32),
            pltpu.VMEM((V,), jnp.float32),
            pltpu.VMEM((CH, VP), jnp.float32),
            pltpu.VMEM((CH, VP), jnp.float32),
            pltpu.VMEM((L,), jnp.float32),
            pltpu.SemaphoreType.DMA,
            pltpu.SemaphoreType.DMA,
            pltpu.SemaphoreType.DMA,
        ],
    )
    return f(table3, idx_flat, tgt_flat, lse)


TB = 512


def _tr_body(in_ref, out_ref):
    out_ref[...] = jnp.transpose(in_ref[...], (1, 0))


def _tr_body_alias(prev_ref, in_ref, out_ref):
    del prev_ref
    out_ref[...] = jnp.transpose(in_ref[...], (1, 0))


def _transpose_half(prev, half_rm, k):
    base = k * H // TB
    out_spec = pl.BlockSpec((V, TB), lambda g, b=base: (0, b + g))
    out_shape = jax.ShapeDtypeStruct((V, N), jnp.float32)
    if prev is None:
        return pl.pallas_call(
            _tr_body,
            grid=(H // TB,),
            in_specs=[pl.BlockSpec((TB, V), lambda g: (g, 0))],
            out_specs=out_spec,
            out_shape=out_shape,
        )(half_rm)
    return pl.pallas_call(
        _tr_body_alias,
        grid=(H // TB,),
        in_specs=[pl.BlockSpec(memory_space=pl.ANY),
                  pl.BlockSpec((TB, V), lambda g: (g, 0))],
        out_specs=out_spec,
        out_shape=out_shape,
        input_output_aliases={0: 0},
    )(prev, half_rm)


def _red_body(part_ref, out_ref):
    out_ref[...] = jnp.sum(part_ref[...], keepdims=True).reshape(1, 1) * (
        1.0 / N)


def _reduce_loss(part):
    return pl.pallas_call(
        _red_body,
        out_shape=jax.ShapeDtypeStruct((1, 1), jnp.float32),
    )(part.reshape(NH * 4, 128))


def kernel(idx, targets, table):
    idx_flat = idx.reshape(N).astype(jnp.int32)
    tgt_flat = targets.reshape(N).astype(jnp.int32)
    table3 = jnp.pad(table, ((0, 0), (0, VP - V)))
    lse = _compute_lse(table).reshape(V)
    halves = []
    parts = []
    for k in range(NH):
        half_rm, p = _sc_gather(table3, idx_flat[k * H:(k + 1) * H],
                                tgt_flat[k * H:(k + 1) * H], lse)
        halves.append(half_rm)
        parts.append(p)
    acc = _transpose_half(None, halves[0], 0)
    for k in range(1, NH):
        acc = _transpose_half(acc, halves[k], k)
    logits_flat = jnp.transpose(acc, (1, 0))
    loss = _reduce_loss(jnp.concatenate(parts))[0, 0]
    return (logits_flat, loss)
